# baseline (speedup 1.0000x reference)
import functools
import jax
import jax.numpy as jnp
from jax import lax
from jax.experimental import pallas as pl
from jax.experimental.pallas import tpu as pltpu
from jax.experimental.pallas import tpu_sc as plsc

F_SIMP = 4096
F_ORIG = 8192
ROWPAD = 16
FEAT = 8
JBLK = 1024
CHUNK = 128
NLANE = 16


def _sc_body(vs_ref, g0_ref, g1_ref, g2_ref, vo_ref, f0_ref, f1_ref, f2_ref,
             a_out, b_out, idx0, idx1, idx2, r0, r1, r2, feat,
             sem0, sem1, sem2):
    wid = lax.axis_index("s") * 2 + lax.axis_index("c")
    lane = lax.iota(jnp.int32, NLANE)
    ones = jnp.full((NLANE,), 1.0, jnp.float32)
    zeros = jnp.full((NLANE,), 0.0, jnp.float32)

    def cvec(c):
        return jnp.full((NLANE,), c, jnp.int32)

    for g in range(CHUNK // NLANE):
        ridx = g * NLANE + lane
        for c in range(5, FEAT):
            plsc.store_scatter(feat, [ridx, cvec(c)], zeros)

    def do_chunk(tbl, fa, fb, fc, out, base, is_a):
        pltpu.sync_copy(fa.at[pl.ds(base, CHUNK)], idx0)
        pltpu.sync_copy(fb.at[pl.ds(base, CHUNK)], idx1)
        pltpu.sync_copy(fc.at[pl.ds(base, CHUNK)], idx2)
        cp0 = pltpu.async_copy(tbl.at[idx0], r0, sem0)
        cp1 = pltpu.async_copy(tbl.at[idx1], r1, sem1)
        cp2 = pltpu.async_copy(tbl.at[idx2], r2, sem2)
        cp0.wait()
        cp1.wait()
        cp2.wait()

        for g in range(CHUNK // NLANE):
            ridx = g * NLANE + lane

            def coord(c):
                ci = cvec(c)
                s = (plsc.load_gather(r0, [ridx, ci])
                     + plsc.load_gather(r1, [ridx, ci])
                     + plsc.load_gather(r2, [ridx, ci]))
                return s * (1.0 / 3.0)

            x, y, z = coord(0), coord(1), coord(2)
            n2 = x * x + y * y + z * z
            if is_a:
                plsc.store_scatter(feat, [ridx, cvec(0)], x)
                plsc.store_scatter(feat, [ridx, cvec(1)], y)
                plsc.store_scatter(feat, [ridx, cvec(2)], z)
                plsc.store_scatter(feat, [ridx, cvec(3)], n2)
                plsc.store_scatter(feat, [ridx, cvec(4)], ones)
            else:
                plsc.store_scatter(feat, [ridx, cvec(0)], -2.0 * x)
                plsc.store_scatter(feat, [ridx, cvec(1)], -2.0 * y)
                plsc.store_scatter(feat, [ridx, cvec(2)], -2.0 * z)
                plsc.store_scatter(feat, [ridx, cvec(3)], ones)
                plsc.store_scatter(feat, [ridx, cvec(4)], n2)

        pltpu.sync_copy(feat, out.at[pl.ds(base, CHUNK)])

    do_chunk(vs_ref, g0_ref, g1_ref, g2_ref, a_out, wid * CHUNK, True)
    do_chunk(vo_ref, f0_ref, f1_ref, f2_ref, b_out, wid * 2 * CHUNK, False)
    do_chunk(vo_ref, f0_ref, f1_ref, f2_ref, b_out, (wid * 2 + 1) * CHUNK, False)


def _sc_features(vs_pad, g0, g1, g2, vo_pad, f0, f1, f2):
    mesh = plsc.VectorSubcoreMesh(core_axis_name="c", subcore_axis_name="s")
    fn = pl.kernel(
        _sc_body,
        out_type=(
            jax.ShapeDtypeStruct((F_SIMP, FEAT), jnp.float32),
            jax.ShapeDtypeStruct((F_ORIG, FEAT), jnp.float32),
        ),
        mesh=mesh,
        compiler_params=pltpu.CompilerParams(
            needs_layout_passes=False, use_tc_tiling_on_sc=False),
        scratch_types=[
            pltpu.VMEM((CHUNK,), jnp.int32),
            pltpu.VMEM((CHUNK,), jnp.int32),
            pltpu.VMEM((CHUNK,), jnp.int32),
            pltpu.VMEM((CHUNK, ROWPAD), jnp.float32),
            pltpu.VMEM((CHUNK, ROWPAD), jnp.float32),
            pltpu.VMEM((CHUNK, ROWPAD), jnp.float32),
            pltpu.VMEM((CHUNK, FEAT), jnp.float32),
            pltpu.SemaphoreType.DMA,
            pltpu.SemaphoreType.DMA,
            pltpu.SemaphoreType.DMA,
        ],
    )
    return fn(vs_pad, g0, g1, g2, vo_pad, f0, f1, f2)


def _tc_body(a_ref, b_ref, p_ref, out_ref, acc_ref):
    j = pl.program_id(0)
    nj = pl.num_programs(0)
    g = lax.dot_general(
        a_ref[...], b_ref[...],
        (((1,), (1,)), ((), ())),
        preferred_element_type=jnp.float32,
        precision=lax.Precision.HIGHEST,
    )
    m = jnp.min(g, axis=1, keepdims=True)

    @pl.when(j == 0)
    def _():
        acc_ref[...] = m

    @pl.when(j > 0)
    def _():
        acc_ref[...] = jnp.minimum(acc_ref[...], m)

    @pl.when(j == nj - 1)
    def _():
        out_ref[...] = jnp.sum(acc_ref[...] * p_ref[...], keepdims=True)


def _tc_min_loss(a_feat, b_feat, probs):
    grid = (F_ORIG // JBLK,)
    return pl.pallas_call(
        _tc_body,
        grid=grid,
        in_specs=[
            pl.BlockSpec((F_SIMP, FEAT), lambda j: (0, 0)),
            pl.BlockSpec((JBLK, FEAT), lambda j: (j, 0)),
            pl.BlockSpec((F_SIMP, 1), lambda j: (0, 0)),
        ],
        out_specs=pl.BlockSpec((1, 1), lambda j: (0, 0)),
        out_shape=jax.ShapeDtypeStruct((1, 1), jnp.float32),
        scratch_shapes=[pltpu.VMEM((F_SIMP, 1), jnp.float32)],
    )(a_feat, b_feat, probs)


def kernel(original_vertices, original_faces, simplified_vertices,
           simplified_faces, face_probabilities):
    of = original_faces.astype(jnp.int32)
    sf = simplified_faces.astype(jnp.int32)
    vo_pad = jnp.pad(original_vertices, ((0, 0), (0, ROWPAD - 3)))
    vs_pad = jnp.pad(simplified_vertices, ((0, 0), (0, ROWPAD - 3)))
    a_feat, b_feat = _sc_features(
        vs_pad, sf[:, 0], sf[:, 1], sf[:, 2],
        vo_pad, of[:, 0], of[:, 1], of[:, 2])
    loss = _tc_min_loss(a_feat, b_feat, face_probabilities.reshape(F_SIMP, 1))
    return loss[0, 0]
